# Initial kernel scaffold; baseline (speedup 1.0000x reference)
import jax
import jax.numpy as jnp
from jax.experimental import pallas as pl


def kernel(vertices, faces):
    raise NotImplementedError("write your pallas kernel here")


# speedup vs baseline: 4.1614x; 4.1614x over previous
import functools

import numpy as np
import jax
import jax.numpy as jnp
from jax import lax
from jax.experimental import pallas as pl
from jax.experimental.pallas import tpu as pltpu
from jax.experimental.pallas import tpu_sc as plsc

NS = 5
KNN = 8
NV = 2500
NF = 4000
FP = 4096
NP = NF * NS
PT = 256
NPP = 20480
GRID = NPP // PT

_FACEID = np.arange(FP, dtype=np.float32)
_PADINF = np.where(np.arange(FP) < NF, 0.0, np.inf).astype(np.float32)
_pidx = np.arange(NPP)
_PFI = np.where(_pidx < NP, _pidx // NS, -1).astype(np.float32)
_VALID = (_pidx < NP).astype(np.float32)


_CHUNKS = 3


def _gather_tri_rows(table, idx):
    mesh = plsc.VectorSubcoreMesh(core_axis_name="c", subcore_axis_name="s")

    @functools.partial(
        pl.kernel,
        mesh=mesh,
        out_type=jax.ShapeDtypeStruct((12288, 128), jnp.float32),
        scratch_types=[
            pltpu.VMEM((_CHUNKS, 128), jnp.int32),
            pltpu.VMEM((_CHUNKS * 128, 128), jnp.float32),
            pltpu.SemaphoreType.DMA,
        ],
    )
    def k(table_hbm, idx_hbm, out_hbm, idx_v, rows_v, sem):
        wid = lax.axis_index("s") * 2 + lax.axis_index("c")
        base = wid * (_CHUNKS * 128)
        for j in range(_CHUNKS):
            pltpu.sync_copy(idx_hbm.at[pl.ds(base + j * 128, 128)], idx_v.at[j])
            pltpu.async_copy(table_hbm.at[idx_v.at[j]],
                             rows_v.at[pl.ds(j * 128, 128)], sem).wait()
        pltpu.sync_copy(rows_v, out_hbm.at[pl.ds(base, _CHUNKS * 128)])

    return k(table, idx)


def _main_body(pp_ref, c_ref, out_ref):
    i = pl.program_id(0)
    pp = pp_ref[...]
    px, py, pz = pp[:, 0:1], pp[:, 1:2], pp[:, 2:3]
    pfi, valid = pp[:, 3:4], pp[:, 4:5]
    c = c_ref[...]
    cx, cy, cz = c[0:1], c[1:2], c[2:3]
    n0, n1, n2 = c[3:4], c[4:5], c[5:6]
    a00, bb, cc, d01 = c[6:7], c[7:8], c[8:9], c[9:10]
    denomp, area = c[10:11], c[11:12]
    faceid, padinf = c[12:13], c[13:14]
    v0x, v0y, v0z = c[15:16], c[16:17], c[17:18]

    dx, dy, dz = px - cx, py - cy, pz - cz
    d2 = ((dx * dx + dy * dy) + dz * dz) + padinf

    dot0p = (n0 * (px - v0x) + n1 * (py - v0y)) + n2 * (pz - v0z)
    nu = a00 * dot0p - bb
    nv = cc - d01 * dot0p
    inside = (nu >= 0.0) & (nv >= 0.0) & ((nu + nv) <= denomp)
    keep = inside & (faceid != pfi)
    contrib = jnp.where(keep, area, 0.0) * valid

    work = d2
    member = jnp.zeros(d2.shape, dtype=jnp.bool_)
    for _ in range(KNN):
        m = jnp.min(work, axis=1, keepdims=True)
        eq = work == m
        member = member | eq
        work = jnp.where(eq, jnp.inf, work)

    s = jnp.sum(jnp.where(member, contrib, 0.0), axis=1, keepdims=True)

    @pl.when(i == 0)
    def _():
        out_ref[...] = jnp.zeros((PT, 1), jnp.float32)

    out_ref[...] += s


def _main_call(pp, consts):
    return pl.pallas_call(
        _main_body,
        grid=(GRID,),
        in_specs=[
            pl.BlockSpec((PT, 8), lambda i: (i, 0)),
            pl.BlockSpec((24, FP), lambda i: (0, 0)),
        ],
        out_specs=pl.BlockSpec((PT, 1), lambda i: (0, 0)),
        out_shape=jax.ShapeDtypeStruct((PT, 1), jnp.float32),
    )(pp, consts)


def _padf(x):
    return jnp.pad(x, (0, FP - NF))


def kernel(vertices, faces):
    vpad = jnp.pad(vertices, ((0, 0), (0, 125)))
    fidx = jnp.pad(faces.reshape(-1), (0, 12288 - 3 * NF))
    rows = _gather_tri_rows(vpad, fidx)
    tri = rows[:3 * NF, :3].reshape(NF, 3, 3)
    v0, v1, v2 = tri[:, 0, :], tri[:, 1, :], tri[:, 2, :]

    rkey = jax.random.key(1)
    ku, kv = jax.random.split(rkey)
    u = jax.random.uniform(ku, (NF, NS, 1), dtype=jnp.float32)
    v = jax.random.uniform(kv, (NF, NS, 1), dtype=jnp.float32)
    mask = (u + v) > 1.0
    u = jnp.where(mask, 1.0 - u, u)
    v = jnp.where(mask, 1.0 - v, v)
    w = 1.0 - u - v
    points = v0[:, None, :] * w + v1[:, None, :] * u + v2[:, None, :] * v
    points = points.reshape(-1, 3)

    centroids = tri.mean(axis=1)

    edge1 = v1 - v0
    edge2 = v2 - v0
    normals = jnp.cross(edge1, edge2)
    normal_lengths = jnp.linalg.norm(normals, axis=1, keepdims=True)
    normals = normals / (normal_lengths + 1e-08)
    dot00 = jnp.sum(normals * normals, axis=1)
    dot01 = jnp.sum(normals * edge1, axis=1)
    dot02 = jnp.sum(normals * edge2, axis=1)
    denom = dot00 * dot00 - dot01 * dot01
    denomp = denom + 1e-08
    bb = dot01 * dot02
    cc = dot00 * dot02
    cross_prod = jnp.cross(edge1, edge2)
    tri_areas = 0.5 * jnp.linalg.norm(cross_prod, axis=1)

    zrow = jnp.zeros((FP,), jnp.float32)
    consts = jnp.stack(
        [_padf(centroids[:, 0]), _padf(centroids[:, 1]), _padf(centroids[:, 2]),
         _padf(normals[:, 0]), _padf(normals[:, 1]), _padf(normals[:, 2]),
         _padf(dot00), _padf(bb), _padf(cc), _padf(dot01),
         _padf(denomp), _padf(tri_areas),
         jnp.asarray(_FACEID), jnp.asarray(_PADINF), zrow,
         _padf(v0[:, 0]), _padf(v0[:, 1]), _padf(v0[:, 2]),
         zrow, zrow, zrow, zrow, zrow, zrow], axis=0)

    padp = jnp.zeros((NPP - NP,), jnp.float32)
    zcol = jnp.zeros((NPP,), jnp.float32)
    pp = jnp.stack(
        [jnp.concatenate([points[:, 0], padp]),
         jnp.concatenate([points[:, 1], padp]),
         jnp.concatenate([points[:, 2], padp]),
         jnp.asarray(_PFI), jnp.asarray(_VALID),
         zcol, zcol, zcol], axis=1)

    out = _main_call(pp, consts)
    return jnp.sum(out)


# speedup vs baseline: 5.3186x; 1.2781x over previous
import functools

import numpy as np
import jax
import jax.numpy as jnp
from jax import lax
from jax.experimental import pallas as pl
from jax.experimental.pallas import tpu as pltpu
from jax.experimental.pallas import tpu_sc as plsc

NS = 5
KNN = 8
NV = 2500
NF = 4000
FP = 4096
NP = NF * NS
PT = 512
NPP = 20480
GRID = NPP // PT

_FACEID = np.arange(FP, dtype=np.float32)
_PADINF = np.where(np.arange(FP) < NF, 0.0, np.inf).astype(np.float32)
_pidx = np.arange(NPP)
_PFI = np.where(_pidx < NP, _pidx // NS, -1).astype(np.float32)
_VALID = (_pidx < NP).astype(np.float32)


_CHUNKS = 3


def _gather_tri_rows(table, idx):
    mesh = plsc.VectorSubcoreMesh(core_axis_name="c", subcore_axis_name="s")

    @functools.partial(
        pl.kernel,
        mesh=mesh,
        out_type=jax.ShapeDtypeStruct((12288, 128), jnp.float32),
        scratch_types=[
            pltpu.VMEM((_CHUNKS, 128), jnp.int32),
            pltpu.VMEM((_CHUNKS * 128, 128), jnp.float32),
            pltpu.SemaphoreType.DMA,
        ],
    )
    def k(table_hbm, idx_hbm, out_hbm, idx_v, rows_v, sem):
        wid = lax.axis_index("s") * 2 + lax.axis_index("c")
        base = wid * (_CHUNKS * 128)
        for j in range(_CHUNKS):
            pltpu.sync_copy(idx_hbm.at[pl.ds(base + j * 128, 128)], idx_v.at[j])
            pltpu.async_copy(table_hbm.at[idx_v.at[j]],
                             rows_v.at[pl.ds(j * 128, 128)], sem).wait()
        pltpu.sync_copy(rows_v, out_hbm.at[pl.ds(base, _CHUNKS * 128)])

    return k(table, idx)


def _main_body(pp_ref, c_ref, out_ref):
    i = pl.program_id(0)
    pp = pp_ref[...]
    px, py, pz = pp[:, 0:1], pp[:, 1:2], pp[:, 2:3]
    pfi, valid = pp[:, 3:4], pp[:, 4:5]
    c = c_ref[...]
    cx, cy, cz = c[0:1], c[1:2], c[2:3]
    n0, n1, n2 = c[3:4], c[4:5], c[5:6]
    a00, bb, cc, d01 = c[6:7], c[7:8], c[8:9], c[9:10]
    denomp, area = c[10:11], c[11:12]
    faceid, padinf = c[12:13], c[13:14]
    v0x, v0y, v0z = c[15:16], c[16:17], c[17:18]

    dx, dy, dz = px - cx, py - cy, pz - cz
    d2 = ((dx * dx + dy * dy) + dz * dz) + padinf

    work = d2
    for _ in range(KNN):
        m = jnp.min(work, axis=1, keepdims=True)
        work = jnp.where(work == m, jnp.inf, work)
    member = work != d2

    dot0p = (n0 * (px - v0x) + n1 * (py - v0y)) + n2 * (pz - v0z)
    nu = a00 * dot0p - bb
    nv = cc - d01 * dot0p
    inside = (nu >= 0.0) & (nv >= 0.0) & ((nu + nv) <= denomp)
    keep = inside & (faceid != pfi) & member
    contrib = jnp.where(keep, area, 0.0)

    s = jnp.sum(contrib, axis=1, keepdims=True) * valid

    @pl.when(i == 0)
    def _():
        out_ref[...] = jnp.zeros((PT, 1), jnp.float32)

    out_ref[...] += s


def _main_call(pp, consts):
    return pl.pallas_call(
        _main_body,
        grid=(GRID,),
        in_specs=[
            pl.BlockSpec((PT, 8), lambda i: (i, 0)),
            pl.BlockSpec((24, FP), lambda i: (0, 0)),
        ],
        out_specs=pl.BlockSpec((PT, 1), lambda i: (0, 0)),
        out_shape=jax.ShapeDtypeStruct((PT, 1), jnp.float32),
    )(pp, consts)


def _padf(x):
    return jnp.pad(x, (0, FP - NF))


def kernel(vertices, faces):
    vpad = jnp.pad(vertices, ((0, 0), (0, 125)))
    fidx = jnp.pad(faces.reshape(-1), (0, 12288 - 3 * NF))
    rows = _gather_tri_rows(vpad, fidx)
    tri = rows[:3 * NF, :3].reshape(NF, 3, 3)
    v0, v1, v2 = tri[:, 0, :], tri[:, 1, :], tri[:, 2, :]

    rkey = jax.random.key(1)
    ku, kv = jax.random.split(rkey)
    u = jax.random.uniform(ku, (NF, NS, 1), dtype=jnp.float32)
    v = jax.random.uniform(kv, (NF, NS, 1), dtype=jnp.float32)
    mask = (u + v) > 1.0
    u = jnp.where(mask, 1.0 - u, u)
    v = jnp.where(mask, 1.0 - v, v)
    w = 1.0 - u - v
    points = v0[:, None, :] * w + v1[:, None, :] * u + v2[:, None, :] * v
    points = points.reshape(-1, 3)

    centroids = tri.mean(axis=1)

    edge1 = v1 - v0
    edge2 = v2 - v0
    normals = jnp.cross(edge1, edge2)
    normal_lengths = jnp.linalg.norm(normals, axis=1, keepdims=True)
    normals = normals / (normal_lengths + 1e-08)
    dot00 = jnp.sum(normals * normals, axis=1)
    dot01 = jnp.sum(normals * edge1, axis=1)
    dot02 = jnp.sum(normals * edge2, axis=1)
    denom = dot00 * dot00 - dot01 * dot01
    denomp = denom + 1e-08
    bb = dot01 * dot02
    cc = dot00 * dot02
    cross_prod = jnp.cross(edge1, edge2)
    tri_areas = 0.5 * jnp.linalg.norm(cross_prod, axis=1)

    zrow = jnp.zeros((FP,), jnp.float32)
    consts = jnp.stack(
        [_padf(centroids[:, 0]), _padf(centroids[:, 1]), _padf(centroids[:, 2]),
         _padf(normals[:, 0]), _padf(normals[:, 1]), _padf(normals[:, 2]),
         _padf(dot00), _padf(bb), _padf(cc), _padf(dot01),
         _padf(denomp), _padf(tri_areas),
         jnp.asarray(_FACEID), jnp.asarray(_PADINF), zrow,
         _padf(v0[:, 0]), _padf(v0[:, 1]), _padf(v0[:, 2]),
         zrow, zrow, zrow, zrow, zrow, zrow], axis=0)

    padp = jnp.zeros((NPP - NP,), jnp.float32)
    zcol = jnp.zeros((NPP,), jnp.float32)
    pp = jnp.stack(
        [jnp.concatenate([points[:, 0], padp]),
         jnp.concatenate([points[:, 1], padp]),
         jnp.concatenate([points[:, 2], padp]),
         jnp.asarray(_PFI), jnp.asarray(_VALID),
         zcol, zcol, zcol], axis=1)

    out = _main_call(pp, consts)
    return jnp.sum(out)
